# Initial kernel scaffold; baseline (speedup 1.0000x reference)
import jax
import jax.numpy as jnp
from jax.experimental import pallas as pl


def kernel(inputs, edge_index, W, b, W1, b1, W2, b2):
    raise NotImplementedError("write your pallas kernel here")


# speedup vs baseline: 3.9014x; 3.9014x over previous
import functools

import jax
import jax.numpy as jnp
from jax import lax
from jax.experimental import pallas as pl
from jax.experimental.pallas import tpu as pltpu
from jax.experimental.pallas import tpu_sc as plsc

NC = 2
NS = 16
K = 80

_mesh = plsc.VectorSubcoreMesh(core_axis_name="c", subcore_axis_name="s")


def _deg_kernels(n, e):
    ept = e // (NC * NS)
    ch = ept // K
    rt = n // NS

    @functools.partial(
        pl.kernel,
        out_type=(
            jax.ShapeDtypeStruct((NC * n,), jnp.float32),
            jax.ShapeDtypeStruct((NC * n,), jnp.float32),
        ),
        mesh=_mesh,
        scratch_types=[
            pltpu.VMEM((K,), jnp.int32),
            pltpu.VMEM((K,), jnp.int32),
            pltpu.VMEM((K,), jnp.float32),
            pltpu.VMEM((rt,), jnp.float32),
            pltpu.VMEM_SHARED((n,), jnp.float32),
            pltpu.VMEM_SHARED((n,), jnp.float32),
        ],
    )
    def deg(src_hbm, dst_hbm, out_s, out_d,
            sidx, didx, ones_v, zv, deg_s, deg_d):
        c = lax.axis_index("c")
        s = lax.axis_index("s")
        one16 = jnp.ones((16,), jnp.float32)
        zero16 = jnp.zeros((16,), jnp.float32)
        for j in range(K // 16):
            ones_v[pl.ds(j * 16, 16)] = one16
        for j in range(rt // 16):
            zv[pl.ds(j * 16, 16)] = zero16
        r0 = s * rt
        pltpu.sync_copy(zv, deg_s.at[pl.ds(r0, rt)])
        pltpu.sync_copy(zv, deg_d.at[pl.ds(r0, rt)])
        plsc.subcore_barrier()
        ebase = (c * NS + s) * ept

        def chunk(i, carry):
            base = ebase + i * K
            pltpu.sync_copy(src_hbm.at[pl.ds(base, K)], sidx)
            pltpu.sync_copy(dst_hbm.at[pl.ds(base, K)], didx)
            pltpu.sync_copy(ones_v, deg_s.at[sidx], add=True)
            pltpu.sync_copy(ones_v, deg_d.at[didx], add=True)
            return carry

        lax.fori_loop(0, ch, chunk, 0)
        plsc.subcore_barrier()
        pltpu.sync_copy(deg_s.at[pl.ds(r0, rt)], out_s.at[pl.ds(c * n + r0, rt)])
        pltpu.sync_copy(deg_d.at[pl.ds(r0, rt)], out_d.at[pl.ds(c * n + r0, rt)])

    return deg


def _agg_kernel(n, e, h):
    ept = e // (NC * NS)
    ch = ept // K
    rt = n // NS

    @functools.partial(
        pl.kernel,
        out_type=jax.ShapeDtypeStruct((NC, n, h), jnp.float32),
        mesh=_mesh,
        scratch_types=[
            pltpu.VMEM((K,), jnp.int32),
            pltpu.VMEM((K,), jnp.int32),
            pltpu.VMEM((K, h), jnp.float32),
            pltpu.SemaphoreType.DMA,
            pltpu.VMEM_SHARED((n, h), jnp.float32),
        ],
    )
    def agg(h_hbm, src_hbm, dst_hbm, zeros_hbm, out,
            sidx, didx, rows, sem, acc):
        c = lax.axis_index("c")
        s = lax.axis_index("s")
        r0 = s * rt
        pltpu.sync_copy(zeros_hbm.at[pl.ds(r0, rt)], acc.at[pl.ds(r0, rt)])
        plsc.subcore_barrier()
        ebase = (c * NS + s) * ept

        def chunk(i, carry):
            base = ebase + i * K
            pltpu.sync_copy(src_hbm.at[pl.ds(base, K)], sidx)
            pltpu.sync_copy(dst_hbm.at[pl.ds(base, K)], didx)
            pltpu.async_copy(h_hbm.at[sidx], rows, sem).wait()
            pltpu.sync_copy(rows, acc.at[didx], add=True)
            return carry

        lax.fori_loop(0, ch, chunk, 0)
        plsc.subcore_barrier()
        pltpu.sync_copy(acc.at[pl.ds(r0, rt)], out.at[c, pl.ds(r0, rt)])

    return agg


def _mm_body(x_ref, w_ref, d0_ref, d1_ref, h_ref):
    deg = d0_ref[...] + d1_ref[...]
    nsrc = lax.rsqrt(jnp.maximum(deg, 1.0))
    xs = x_ref[...] * nsrc
    h_ref[...] = jnp.dot(xs, w_ref[...], preferred_element_type=jnp.float32)


def _fin_body(p_ref, d0_ref, d1_ref, b_ref, w1_ref, b1_ref, w2_ref, b2_ref,
              o_ref):
    n = d0_ref.shape[0]
    agg = p_ref[0, 0:n, :] + p_ref[1, 0:n, :]
    deg = d0_ref[...] + d1_ref[...]
    ndst = lax.rsqrt(jnp.maximum(deg, 1.0))
    hrelu = jnp.maximum(agg * ndst + b_ref[...], 0.0)
    hg = jnp.sum(hrelu, axis=0, keepdims=True) * (1.0 / n)
    o1 = jnp.dot(hg, w1_ref[...], preferred_element_type=jnp.float32)
    o1 = o1 + b1_ref[...]
    o2 = jnp.dot(o1, w2_ref[...], preferred_element_type=jnp.float32)
    o_ref[...] = o2 + b2_ref[...]


def kernel(inputs, edge_index, W, b, W1, b1, W2, b2):
    n, d = inputs.shape
    h = W.shape[1]
    e = edge_index.shape[1]
    npad = -(-n // (NS * 16)) * (NS * 16)
    src = edge_index[0]
    dst = edge_index[1]

    zeros_h = jnp.zeros((npad, h), jnp.float32)

    deg_s, deg_d = _deg_kernels(npad, e)(src, dst)
    deg_s = deg_s.reshape(NC, npad)
    deg_d = deg_d.reshape(NC, npad)

    hmat = pl.pallas_call(
        _mm_body,
        out_shape=jax.ShapeDtypeStruct((n, h), jnp.float32),
    )(inputs, W, deg_s[0, :n, None], deg_s[1, :n, None])

    parts = _agg_kernel(npad, e, h)(hmat, src, dst, zeros_h)

    out = pl.pallas_call(
        _fin_body,
        out_shape=jax.ShapeDtypeStruct((1, W2.shape[1]), jnp.float32),
    )(parts, deg_d[0, :n, None], deg_d[1, :n, None], b.reshape(1, h),
      W1, b1.reshape(1, -1), W2, b2.reshape(1, -1))
    return out


# speedup vs baseline: 6.9646x; 1.7851x over previous
import functools

import jax
import jax.numpy as jnp
from jax import lax
from jax.experimental import pallas as pl
from jax.experimental.pallas import tpu as pltpu
from jax.experimental.pallas import tpu_sc as plsc

NC = 2
NS = 16
K = 80

_mesh = plsc.VectorSubcoreMesh(core_axis_name="c", subcore_axis_name="s")

_ONE16 = functools.partial(jnp.ones, (16,), jnp.float32)
_ZERO16 = functools.partial(jnp.zeros, (16,), jnp.float32)


def _deg_kernel(n, e):
    ept = e // (NC * NS)
    ch = ept // K
    rt = n // NS

    @functools.partial(
        pl.kernel,
        out_type=jax.ShapeDtypeStruct((NC * n,), jnp.float32),
        mesh=_mesh,
        scratch_types=[
            pltpu.VMEM((K,), jnp.int32),
            pltpu.VMEM((K,), jnp.int32),
            pltpu.VMEM((K,), jnp.float32),
            pltpu.VMEM((rt,), jnp.float32),
            pltpu.SemaphoreType.DMA,
            pltpu.SemaphoreType.DMA,
            pltpu.VMEM_SHARED((n,), jnp.float32),
        ],
    )
    def deg(src_hbm, out_s, idx0, idx1, ones_v, zv, isem0, isem1, deg_s):
        c = lax.axis_index("c")
        s = lax.axis_index("s")
        for j in range(K // 16):
            ones_v[pl.ds(j * 16, 16)] = _ONE16()
        for j in range(rt // 16):
            zv[pl.ds(j * 16, 16)] = _ZERO16()
        r0 = s * rt
        pltpu.sync_copy(zv, deg_s.at[pl.ds(r0, rt)])
        plsc.subcore_barrier()
        ebase = (c * NS + s) * ept

        idx = (idx0, idx1)
        isem = (isem0, isem1)
        pltpu.sync_copy(src_hbm.at[pl.ds(ebase, K)], idx0)
        pltpu.async_copy(src_hbm.at[pl.ds(ebase + K, K)], idx1, isem1)

        def step(i, carry):
            def body(p, q):
                @pl.when(i + 1 < ch)
                def _():
                    pltpu.make_async_copy(
                        src_hbm.at[pl.ds(ebase + (i + 1) * K, K)],
                        idx[q], isem[q]).wait()
                pltpu.sync_copy(ones_v, deg_s.at[idx[p]], add=True)

                @pl.when(i + 2 < ch)
                def _():
                    pltpu.async_copy(
                        src_hbm.at[pl.ds(ebase + (i + 2) * K, K)],
                        idx[p], isem[p])

            even = lax.rem(i, 2) == 0

            @pl.when(even)
            def _():
                body(0, 1)

            @pl.when(jnp.logical_not(even))
            def _():
                body(1, 0)

            return carry

        lax.fori_loop(0, ch, step, 0)
        plsc.subcore_barrier()
        pltpu.sync_copy(deg_s.at[pl.ds(r0, rt)], out_s.at[pl.ds(c * n + r0, rt)])

    return deg


def _agg_kernel(n, e, h):
    ept = e // (NC * NS)
    ch = ept // K
    rt = n // NS

    @functools.partial(
        pl.kernel,
        out_type=(
            jax.ShapeDtypeStruct((NC, n, h), jnp.float32),
            jax.ShapeDtypeStruct((NC * n,), jnp.float32),
        ),
        mesh=_mesh,
        scratch_types=[
            pltpu.VMEM((K,), jnp.int32),
            pltpu.VMEM((K,), jnp.int32),
            pltpu.VMEM((K,), jnp.int32),
            pltpu.VMEM((K,), jnp.int32),
            pltpu.VMEM((K, h), jnp.float32),
            pltpu.VMEM((K, h), jnp.float32),
            pltpu.VMEM((K,), jnp.float32),
            pltpu.VMEM((rt,), jnp.float32),
            pltpu.SemaphoreType.DMA,
            pltpu.SemaphoreType.DMA,
            pltpu.SemaphoreType.DMA,
            pltpu.SemaphoreType.DMA,
            pltpu.VMEM_SHARED((n, h), jnp.float32),
            pltpu.VMEM_SHARED((n,), jnp.float32),
        ],
    )
    def agg(h_hbm, src_hbm, dst_hbm, out, out_d,
            sidx0, sidx1, didx0, didx1, rows0, rows1, ones_v, zv,
            isem0, isem1, gsem0, gsem1, acc, deg_d):
        c = lax.axis_index("c")
        s = lax.axis_index("s")
        for j in range(K // 16):
            ones_v[pl.ds(j * 16, 16)] = _ONE16()
        for j in range(rt // 16):
            zv[pl.ds(j * 16, 16)] = _ZERO16()
        z16 = _ZERO16()
        for r in range(K):
            for j in range(h // 16):
                rows0[r, pl.ds(j * 16, 16)] = z16
        r0 = s * rt
        pltpu.sync_copy(zv, deg_d.at[pl.ds(r0, rt)])
        for j in range(rt // K):
            pltpu.sync_copy(rows0, acc.at[pl.ds(r0 + j * K, K)])
        plsc.subcore_barrier()
        ebase = (c * NS + s) * ept

        sidx = (sidx0, sidx1)
        didx = (didx0, didx1)
        rows = (rows0, rows1)
        isem = (isem0, isem1)
        gsem = (gsem0, gsem1)

        pltpu.sync_copy(src_hbm.at[pl.ds(ebase, K)], sidx0)
        pltpu.sync_copy(dst_hbm.at[pl.ds(ebase, K)], didx0)
        pltpu.async_copy(h_hbm.at[sidx0], rows0, gsem0)
        pltpu.async_copy(src_hbm.at[pl.ds(ebase + K, K)], sidx1, isem1)
        pltpu.async_copy(dst_hbm.at[pl.ds(ebase + K, K)], didx1, isem1)

        def step(i, carry):
            def body(p, q):
                @pl.when(i + 1 < ch)
                def _():
                    b1 = ebase + (i + 1) * K
                    pltpu.make_async_copy(
                        src_hbm.at[pl.ds(b1, K)], sidx[q], isem[q]).wait()
                    pltpu.make_async_copy(
                        dst_hbm.at[pl.ds(b1, K)], didx[q], isem[q]).wait()
                    pltpu.async_copy(h_hbm.at[sidx[q]], rows[q], gsem[q])

                pltpu.make_async_copy(h_hbm.at[sidx[p]], rows[p],
                                      gsem[p]).wait()
                pltpu.sync_copy(rows[p], acc.at[didx[p]], add=True)
                pltpu.sync_copy(ones_v, deg_d.at[didx[p]], add=True)

                @pl.when(i + 2 < ch)
                def _():
                    b2 = ebase + (i + 2) * K
                    pltpu.async_copy(src_hbm.at[pl.ds(b2, K)], sidx[p],
                                     isem[p])
                    pltpu.async_copy(dst_hbm.at[pl.ds(b2, K)], didx[p],
                                     isem[p])

            even = lax.rem(i, 2) == 0

            @pl.when(even)
            def _():
                body(0, 1)

            @pl.when(jnp.logical_not(even))
            def _():
                body(1, 0)

            return carry

        lax.fori_loop(0, ch, step, 0)
        plsc.subcore_barrier()
        pltpu.sync_copy(acc.at[pl.ds(r0, rt)], out.at[c, pl.ds(r0, rt)])
        pltpu.sync_copy(deg_d.at[pl.ds(r0, rt)],
                        out_d.at[pl.ds(c * n + r0, rt)])

    return agg


def _mm_body(x_ref, w_ref, g_ref):
    g_ref[...] = jnp.dot(x_ref[...], w_ref[...],
                         preferred_element_type=jnp.float32)


def _scale_body(g_ref, d0_ref, d1_ref, h_ref):
    deg = d0_ref[...] + d1_ref[...]
    nsrc = lax.rsqrt(jnp.maximum(deg, 1.0))
    h_ref[...] = g_ref[...] * nsrc


def _fin_body(p_ref, d0_ref, d1_ref, b_ref, w1_ref, b1_ref, w2_ref, b2_ref,
              o_ref):
    n = d0_ref.shape[0]
    agg = p_ref[0, 0:n, :] + p_ref[1, 0:n, :]
    deg = d0_ref[...] + d1_ref[...]
    ndst = lax.rsqrt(jnp.maximum(deg, 1.0))
    hrelu = jnp.maximum(agg * ndst + b_ref[...], 0.0)
    hg = jnp.sum(hrelu, axis=0, keepdims=True) * (1.0 / n)
    o1 = jnp.dot(hg, w1_ref[...], preferred_element_type=jnp.float32)
    o1 = o1 + b1_ref[...]
    o2 = jnp.dot(o1, w2_ref[...], preferred_element_type=jnp.float32)
    o_ref[...] = o2 + b2_ref[...]


def kernel(inputs, edge_index, W, b, W1, b1, W2, b2):
    n, d = inputs.shape
    h = W.shape[1]
    e = edge_index.shape[1]
    npad = -(-n // (NS * 16)) * (NS * 16)
    src = edge_index[0]
    dst = edge_index[1]

    gmat = pl.pallas_call(
        _mm_body,
        out_shape=jax.ShapeDtypeStruct((n, h), jnp.float32),
    )(inputs, W)

    deg_s = _deg_kernel(npad, e)(src).reshape(NC, npad)

    hmat = pl.pallas_call(
        _scale_body,
        out_shape=jax.ShapeDtypeStruct((n, h), jnp.float32),
    )(gmat, deg_s[0, :n, None], deg_s[1, :n, None])

    parts, deg_d = _agg_kernel(npad, e, h)(hmat, src, dst)
    deg_d = deg_d.reshape(NC, npad)

    out = pl.pallas_call(
        _fin_body,
        out_shape=jax.ShapeDtypeStruct((1, W2.shape[1]), jnp.float32),
    )(parts, deg_d[0, :n, None], deg_d[1, :n, None], b.reshape(1, h),
      W1, b1.reshape(1, -1), W2, b2.reshape(1, -1))
    return out


# speedup vs baseline: 9.1807x; 1.3182x over previous
import functools

import jax
import jax.numpy as jnp
from jax import lax
from jax.experimental import pallas as pl
from jax.experimental.pallas import tpu as pltpu
from jax.experimental.pallas import tpu_sc as plsc

NC = 2
NS = 16
K = 80

_mesh = plsc.VectorSubcoreMesh(core_axis_name="c", subcore_axis_name="s")


def _zero_fill(ref, words):
    def body(j, carry):
        ref[pl.ds(j * 16, 16)] = jnp.zeros((16,), jnp.float32)
        return carry

    lax.fori_loop(0, words // 16, body, 0)


def _deg_kernel(n, e):
    ept = e // (NC * NS)
    rt = n // NS

    @functools.partial(
        pl.kernel,
        out_type=(
            jax.ShapeDtypeStruct((NC * n,), jnp.float32),
            jax.ShapeDtypeStruct((NC * n,), jnp.float32),
        ),
        mesh=_mesh,
        compiler_params=pltpu.CompilerParams(needs_layout_passes=False),
        scratch_types=[
            pltpu.VMEM((ept,), jnp.int32),
            pltpu.VMEM((ept,), jnp.int32),
            pltpu.VMEM((n,), jnp.float32),
            pltpu.VMEM((n,), jnp.float32),
            pltpu.VMEM((rt,), jnp.float32),
            pltpu.VMEM((rt,), jnp.float32),
            pltpu.SemaphoreType.DMA,
            pltpu.VMEM_SHARED((NS, n), jnp.float32),
            pltpu.VMEM_SHARED((NS, n), jnp.float32),
        ],
    )
    def deg(src_hbm, dst_hbm, out_s, out_d,
            sidx, didx, hist_s, hist_d, racc, tmp, isem, sh_s, sh_d):
        c = lax.axis_index("c")
        s = lax.axis_index("s")
        ebase = (c * NS + s) * ept
        pltpu.async_copy(src_hbm.at[pl.ds(ebase, ept)], sidx, isem)
        pltpu.async_copy(dst_hbm.at[pl.ds(ebase, ept)], didx, isem)
        _zero_fill(hist_s, n)
        _zero_fill(hist_d, n)
        pltpu.make_async_copy(src_hbm.at[pl.ds(ebase, ept)], sidx,
                              isem).wait()
        pltpu.make_async_copy(dst_hbm.at[pl.ds(ebase, ept)], didx,
                              isem).wait()
        ones16 = jnp.ones((16,), jnp.float32)

        def hbody(j, carry):
            iv_s = sidx[pl.ds(j * 16, 16)]
            plsc.addupdate_scatter(hist_s, [iv_s], ones16)
            iv_d = didx[pl.ds(j * 16, 16)]
            plsc.addupdate_scatter(hist_d, [iv_d], ones16)
            return carry

        lax.fori_loop(0, ept // 16, hbody, 0)
        pltpu.sync_copy(hist_s, sh_s.at[s])
        pltpu.sync_copy(hist_d, sh_d.at[s])
        plsc.subcore_barrier()
        r0 = s * rt

        def merge(table, out_ref):
            pltpu.sync_copy(table.at[0, pl.ds(r0, rt)], racc)

            def mbody(t, carry):
                pltpu.sync_copy(table.at[t, pl.ds(r0, rt)], tmp)

                def abody(j, c2):
                    sl = pl.ds(j * 16, 16)
                    racc[sl] = racc[sl] + tmp[sl]
                    return c2

                lax.fori_loop(0, rt // 16, abody, 0)
                return carry

            lax.fori_loop(1, NS, mbody, 0)
            pltpu.sync_copy(racc, out_ref)

        merge(sh_s, out_s.at[pl.ds(c * n + r0, rt)])
        merge(sh_d, out_d.at[pl.ds(c * n + r0, rt)])

    return deg


def _agg_kernel(n, e, h):
    ept = e // (NC * NS)
    ch = ept // K
    rt = n // NS

    @functools.partial(
        pl.kernel,
        out_type=jax.ShapeDtypeStruct((NC, n, h), jnp.float32),
        mesh=_mesh,
        scratch_types=[
            pltpu.VMEM((K,), jnp.int32),
            pltpu.VMEM((K,), jnp.int32),
            pltpu.VMEM((K,), jnp.int32),
            pltpu.VMEM((K,), jnp.int32),
            pltpu.VMEM((K, h), jnp.float32),
            pltpu.VMEM((K, h), jnp.float32),
            pltpu.SemaphoreType.DMA,
            pltpu.SemaphoreType.DMA,
            pltpu.SemaphoreType.DMA,
            pltpu.SemaphoreType.DMA,
            pltpu.VMEM_SHARED((n, h), jnp.float32),
        ],
    )
    def agg(h_hbm, src_hbm, dst_hbm, out,
            sidx0, sidx1, didx0, didx1, rows0, rows1,
            isem0, isem1, gsem0, gsem1, acc):
        c = lax.axis_index("c")
        s = lax.axis_index("s")
        z16 = jnp.zeros((16,), jnp.float32)
        for r in range(K):
            for j in range(h // 16):
                rows0[r, pl.ds(j * 16, 16)] = z16
        r0 = s * rt
        for j in range(rt // K):
            pltpu.sync_copy(rows0, acc.at[pl.ds(r0 + j * K, K)])
        plsc.subcore_barrier()
        ebase = (c * NS + s) * ept

        sidx = (sidx0, sidx1)
        didx = (didx0, didx1)
        rows = (rows0, rows1)
        isem = (isem0, isem1)
        gsem = (gsem0, gsem1)

        pltpu.sync_copy(src_hbm.at[pl.ds(ebase, K)], sidx0)
        pltpu.sync_copy(dst_hbm.at[pl.ds(ebase, K)], didx0)
        pltpu.async_copy(h_hbm.at[sidx0], rows0, gsem0)
        pltpu.async_copy(src_hbm.at[pl.ds(ebase + K, K)], sidx1, isem1)
        pltpu.async_copy(dst_hbm.at[pl.ds(ebase + K, K)], didx1, isem1)

        def step(i, carry):
            def body(p, q):
                @pl.when(i + 1 < ch)
                def _():
                    b1 = ebase + (i + 1) * K
                    pltpu.make_async_copy(
                        src_hbm.at[pl.ds(b1, K)], sidx[q], isem[q]).wait()
                    pltpu.make_async_copy(
                        dst_hbm.at[pl.ds(b1, K)], didx[q], isem[q]).wait()
                    pltpu.async_copy(h_hbm.at[sidx[q]], rows[q], gsem[q])

                pltpu.make_async_copy(h_hbm.at[sidx[p]], rows[p],
                                      gsem[p]).wait()
                pltpu.sync_copy(rows[p], acc.at[didx[p]], add=True)

                @pl.when(i + 2 < ch)
                def _():
                    b2 = ebase + (i + 2) * K
                    pltpu.async_copy(src_hbm.at[pl.ds(b2, K)], sidx[p],
                                     isem[p])
                    pltpu.async_copy(dst_hbm.at[pl.ds(b2, K)], didx[p],
                                     isem[p])

            even = lax.rem(i, 2) == 0

            @pl.when(even)
            def _():
                body(0, 1)

            @pl.when(jnp.logical_not(even))
            def _():
                body(1, 0)

            return carry

        lax.fori_loop(0, ch, step, 0)
        plsc.subcore_barrier()
        pltpu.sync_copy(acc.at[pl.ds(r0, rt)], out.at[c, pl.ds(r0, rt)])

    return agg


def _mm_body(x_ref, w_ref, d0_ref, d1_ref, h_ref):
    deg = d0_ref[...] + d1_ref[...]
    nsrc = lax.rsqrt(jnp.maximum(deg, 1.0))
    xs = x_ref[...] * nsrc[:, None]
    h_ref[...] = jnp.dot(xs, w_ref[...], preferred_element_type=jnp.float32)


def _fin_body(p_ref, d0_ref, d1_ref, b_ref, w1_ref, b1_ref, w2_ref, b2_ref,
              o_ref):
    n = d0_ref.shape[0]
    agg = p_ref[0, 0:n, :] + p_ref[1, 0:n, :]
    deg = d0_ref[...] + d1_ref[...]
    ndst = lax.rsqrt(jnp.maximum(deg, 1.0))
    hrelu = jnp.maximum(agg * ndst[:, None] + b_ref[...], 0.0)
    hg = jnp.sum(hrelu, axis=0, keepdims=True) * (1.0 / n)
    o1 = jnp.dot(hg, w1_ref[...], preferred_element_type=jnp.float32)
    o1 = o1 + b1_ref[...]
    o2 = jnp.dot(o1, w2_ref[...], preferred_element_type=jnp.float32)
    o_ref[...] = o2 + b2_ref[...]


def kernel(inputs, edge_index, W, b, W1, b1, W2, b2):
    n, d = inputs.shape
    h = W.shape[1]
    e = edge_index.shape[1]
    npad = -(-n // (NS * 16)) * (NS * 16)
    src = edge_index[0]
    dst = edge_index[1]

    deg_s, deg_d = _deg_kernel(npad, e)(src, dst)

    hmat = pl.pallas_call(
        _mm_body,
        out_shape=jax.ShapeDtypeStruct((n, h), jnp.float32),
    )(inputs, W, deg_s[:n], deg_s[npad:npad + n])

    parts = _agg_kernel(npad, e, h)(hmat, src, dst)

    out = pl.pallas_call(
        _fin_body,
        out_shape=jax.ShapeDtypeStruct((1, W2.shape[1]), jnp.float32),
    )(parts, deg_d[:n], deg_d[npad:npad + n], b.reshape(1, h),
      W1, b1.reshape(1, -1), W2, b2.reshape(1, -1))
    return out


# speedup vs baseline: 10.2441x; 1.1158x over previous
import functools

import jax
import jax.numpy as jnp
from jax import lax
from jax.experimental import pallas as pl
from jax.experimental.pallas import tpu as pltpu
from jax.experimental.pallas import tpu_sc as plsc

NC = 2
NS = 16
K = 80

_mesh = plsc.VectorSubcoreMesh(core_axis_name="c", subcore_axis_name="s")


def _zero_fill(ref, words):
    def body(j, carry):
        ref[pl.ds(j * 16, 16)] = jnp.zeros((16,), jnp.float32)
        return carry

    lax.fori_loop(0, words // 16, body, 0)


def _deg_kernel(n, e):
    ept = e // (NC * NS)
    rt = n // NS

    @functools.partial(
        pl.kernel,
        out_type=(
            jax.ShapeDtypeStruct((NC * n,), jnp.float32),
            jax.ShapeDtypeStruct((NC * n,), jnp.float32),
        ),
        mesh=_mesh,
        compiler_params=pltpu.CompilerParams(needs_layout_passes=False),
        scratch_types=[
            pltpu.VMEM((ept,), jnp.int32),
            pltpu.VMEM((ept,), jnp.int32),
            pltpu.VMEM((n,), jnp.float32),
            pltpu.VMEM((n,), jnp.float32),
            pltpu.VMEM((rt,), jnp.float32),
            pltpu.VMEM((rt,), jnp.float32),
            pltpu.SemaphoreType.DMA,
            pltpu.VMEM_SHARED((NS, n), jnp.float32),
            pltpu.VMEM_SHARED((NS, n), jnp.float32),
        ],
    )
    def deg(src_hbm, dst_hbm, out_s, out_d,
            sidx, didx, hist_s, hist_d, racc, tmp, isem, sh_s, sh_d):
        c = lax.axis_index("c")
        s = lax.axis_index("s")
        ebase = (c * NS + s) * ept
        pltpu.async_copy(src_hbm.at[pl.ds(ebase, ept)], sidx, isem)
        pltpu.async_copy(dst_hbm.at[pl.ds(ebase, ept)], didx, isem)
        _zero_fill(hist_s, n)
        _zero_fill(hist_d, n)
        pltpu.make_async_copy(src_hbm.at[pl.ds(ebase, ept)], sidx,
                              isem).wait()
        pltpu.make_async_copy(dst_hbm.at[pl.ds(ebase, ept)], didx,
                              isem).wait()
        ones16 = jnp.ones((16,), jnp.float32)

        def hbody(j, carry):
            iv_s = sidx[pl.ds(j * 16, 16)]
            plsc.addupdate_scatter(hist_s, [iv_s], ones16)
            iv_d = didx[pl.ds(j * 16, 16)]
            plsc.addupdate_scatter(hist_d, [iv_d], ones16)
            return carry

        lax.fori_loop(0, ept // 16, hbody, 0)
        pltpu.sync_copy(hist_s, sh_s.at[s])
        pltpu.sync_copy(hist_d, sh_d.at[s])
        plsc.subcore_barrier()
        r0 = s * rt

        def merge(table, out_ref):
            pltpu.sync_copy(table.at[0, pl.ds(r0, rt)], racc)

            def mbody(t, carry):
                pltpu.sync_copy(table.at[t, pl.ds(r0, rt)], tmp)

                def abody(j, c2):
                    sl = pl.ds(j * 16, 16)
                    racc[sl] = racc[sl] + tmp[sl]
                    return c2

                lax.fori_loop(0, rt // 16, abody, 0)
                return carry

            lax.fori_loop(1, NS, mbody, 0)
            pltpu.sync_copy(racc, out_ref)

        merge(sh_s, out_s.at[pl.ds(c * n + r0, rt)])
        merge(sh_d, out_d.at[pl.ds(c * n + r0, rt)])

    return deg


def _agg_kernel(n, e, h):
    ept = e // (NC * NS)
    ch = ept // K
    rt = n // NS

    @functools.partial(
        pl.kernel,
        out_type=jax.ShapeDtypeStruct((NC, n, h), jnp.float32),
        mesh=_mesh,
        scratch_types=[
            pltpu.VMEM((K,), jnp.int32),
            pltpu.VMEM((K,), jnp.int32),
            pltpu.VMEM((K,), jnp.int32),
            pltpu.VMEM((K,), jnp.int32),
            pltpu.VMEM((K,), jnp.int32),
            pltpu.VMEM((K,), jnp.int32),
            pltpu.VMEM((K, h), jnp.float32),
            pltpu.VMEM((K, h), jnp.float32),
            pltpu.SemaphoreType.DMA,
            pltpu.SemaphoreType.DMA,
            pltpu.SemaphoreType.DMA,
            pltpu.SemaphoreType.DMA,
            pltpu.SemaphoreType.DMA,
            pltpu.SemaphoreType.DMA,
            pltpu.VMEM_SHARED((n, h), jnp.float32),
        ],
    )
    def agg(h_hbm, src_hbm, dst_hbm, out,
            sidx0, sidx1, didx0, didx1, scp0, scp1, rows0, rows1,
            isem0, isem1, gsem0, gsem1, ssem0, ssem1, acc):
        c = lax.axis_index("c")
        s = lax.axis_index("s")
        z16 = jnp.zeros((16,), jnp.float32)
        for r in range(K):
            for j in range(h // 16):
                rows0[r, pl.ds(j * 16, 16)] = z16
        r0 = s * rt
        for j in range(rt // K):
            pltpu.sync_copy(rows0, acc.at[pl.ds(r0 + j * K, K)])
        plsc.subcore_barrier()
        ebase = (c * NS + s) * ept

        sidx = (sidx0, sidx1)
        didx = (didx0, didx1)
        scp = (scp0, scp1)
        rows = (rows0, rows1)
        isem = (isem0, isem1)
        gsem = (gsem0, gsem1)
        ssem = (ssem0, ssem1)

        pltpu.sync_copy(src_hbm.at[pl.ds(ebase, K)], sidx0)
        pltpu.sync_copy(dst_hbm.at[pl.ds(ebase, K)], didx0)
        pltpu.async_copy(h_hbm.at[sidx0], rows0, gsem0)
        pltpu.async_copy(src_hbm.at[pl.ds(ebase + K, K)], sidx1, isem1)
        pltpu.async_copy(dst_hbm.at[pl.ds(ebase + K, K)], didx1, isem1)

        def step(i, carry):
            def body(p, q):
                @pl.when(i + 1 < ch)
                def _():
                    b1 = ebase + (i + 1) * K
                    pltpu.make_async_copy(
                        src_hbm.at[pl.ds(b1, K)], sidx[q], isem[q]).wait()
                    pltpu.make_async_copy(
                        dst_hbm.at[pl.ds(b1, K)], didx[q], isem[q]).wait()

                    @pl.when(i >= 1)
                    def _():
                        pltpu.make_async_copy(
                            rows[q], acc.at[scp[q]], ssem[q]).wait()

                    pltpu.async_copy(h_hbm.at[sidx[q]], rows[q], gsem[q])

                pltpu.make_async_copy(h_hbm.at[sidx[p]], rows[p],
                                      gsem[p]).wait()
                for j in range(K // 16):
                    sl = pl.ds(j * 16, 16)
                    scp[p][sl] = didx[p][sl]
                pltpu.async_copy(rows[p], acc.at[scp[p]], ssem[p], add=True)

                @pl.when(i + 2 < ch)
                def _():
                    b2 = ebase + (i + 2) * K
                    pltpu.async_copy(src_hbm.at[pl.ds(b2, K)], sidx[p],
                                     isem[p])
                    pltpu.async_copy(dst_hbm.at[pl.ds(b2, K)], didx[p],
                                     isem[p])

            even = lax.rem(i, 2) == 0

            @pl.when(even)
            def _():
                body(0, 1)

            @pl.when(jnp.logical_not(even))
            def _():
                body(1, 0)

            return carry

        lax.fori_loop(0, ch, step, 0)
        lastp = (ch - 1) % 2
        pltpu.make_async_copy(rows[1 - lastp], acc.at[scp[1 - lastp]],
                              ssem[1 - lastp]).wait()
        pltpu.make_async_copy(rows[lastp], acc.at[scp[lastp]],
                              ssem[lastp]).wait()
        plsc.subcore_barrier()
        pltpu.sync_copy(acc.at[pl.ds(r0, rt)], out.at[c, pl.ds(r0, rt)])

    return agg


def _mm_body(x_ref, w_ref, d0_ref, d1_ref, h_ref):
    deg = d0_ref[...] + d1_ref[...]
    nsrc = lax.rsqrt(jnp.maximum(deg, 1.0))
    xs = x_ref[...] * nsrc[:, None]
    h_ref[...] = jnp.dot(xs, w_ref[...], preferred_element_type=jnp.float32)


def _fin_body(p_ref, d0_ref, d1_ref, b_ref, w1_ref, b1_ref, w2_ref, b2_ref,
              o_ref):
    n = d0_ref.shape[0]
    agg = p_ref[0, 0:n, :] + p_ref[1, 0:n, :]
    deg = d0_ref[...] + d1_ref[...]
    ndst = lax.rsqrt(jnp.maximum(deg, 1.0))
    hrelu = jnp.maximum(agg * ndst[:, None] + b_ref[...], 0.0)
    hg = jnp.sum(hrelu, axis=0, keepdims=True) * (1.0 / n)
    o1 = jnp.dot(hg, w1_ref[...], preferred_element_type=jnp.float32)
    o1 = o1 + b1_ref[...]
    o2 = jnp.dot(o1, w2_ref[...], preferred_element_type=jnp.float32)
    o_ref[...] = o2 + b2_ref[...]


def kernel(inputs, edge_index, W, b, W1, b1, W2, b2):
    n, d = inputs.shape
    h = W.shape[1]
    e = edge_index.shape[1]
    npad = -(-n // (NS * 16)) * (NS * 16)
    src = edge_index[0]
    dst = edge_index[1]

    deg_s, deg_d = _deg_kernel(npad, e)(src, dst)

    hmat = pl.pallas_call(
        _mm_body,
        out_shape=jax.ShapeDtypeStruct((n, h), jnp.float32),
    )(inputs, W, deg_s[:n], deg_s[npad:npad + n])

    parts = _agg_kernel(npad, e, h)(hmat, src, dst)

    out = pl.pallas_call(
        _fin_body,
        out_shape=jax.ShapeDtypeStruct((1, W2.shape[1]), jnp.float32),
    )(parts, deg_d[:n], deg_d[npad:npad + n], b.reshape(1, h),
      W1, b1.reshape(1, -1), W2, b2.reshape(1, -1))
    return out


# speedup vs baseline: 10.3927x; 1.0145x over previous
import functools

import jax
import jax.numpy as jnp
from jax import lax
from jax.experimental import pallas as pl
from jax.experimental.pallas import tpu as pltpu
from jax.experimental.pallas import tpu_sc as plsc

NC = 2
NS = 16
K = 128

_mesh = plsc.VectorSubcoreMesh(core_axis_name="c", subcore_axis_name="s")


def _zero_fill(ref, words):
    def body(j, carry):
        ref[pl.ds(j * 16, 16)] = jnp.zeros((16,), jnp.float32)
        return carry

    lax.fori_loop(0, words // 16, body, 0)


def _deg_kernel(n, e):
    ept = e // (NC * NS)
    rt = n // NS

    @functools.partial(
        pl.kernel,
        out_type=(
            jax.ShapeDtypeStruct((NC * n,), jnp.float32),
            jax.ShapeDtypeStruct((NC * n,), jnp.float32),
        ),
        mesh=_mesh,
        compiler_params=pltpu.CompilerParams(needs_layout_passes=False),
        scratch_types=[
            pltpu.VMEM((ept,), jnp.int32),
            pltpu.VMEM((ept,), jnp.int32),
            pltpu.VMEM((n,), jnp.float32),
            pltpu.VMEM((n,), jnp.float32),
            pltpu.VMEM((rt,), jnp.float32),
            pltpu.VMEM((rt,), jnp.float32),
            pltpu.SemaphoreType.DMA,
            pltpu.VMEM_SHARED((NS, n), jnp.float32),
            pltpu.VMEM_SHARED((NS, n), jnp.float32),
        ],
    )
    def deg(src_hbm, dst_hbm, out_s, out_d,
            sidx, didx, hist_s, hist_d, racc, tmp, isem, sh_s, sh_d):
        c = lax.axis_index("c")
        s = lax.axis_index("s")
        ebase = (c * NS + s) * ept
        pltpu.async_copy(src_hbm.at[pl.ds(ebase, ept)], sidx, isem)
        pltpu.async_copy(dst_hbm.at[pl.ds(ebase, ept)], didx, isem)
        _zero_fill(hist_s, n)
        _zero_fill(hist_d, n)
        pltpu.make_async_copy(src_hbm.at[pl.ds(ebase, ept)], sidx,
                              isem).wait()
        pltpu.make_async_copy(dst_hbm.at[pl.ds(ebase, ept)], didx,
                              isem).wait()
        ones16 = jnp.ones((16,), jnp.float32)

        def hbody(j, carry):
            iv_s = sidx[pl.ds(j * 16, 16)]
            plsc.addupdate_scatter(hist_s, [iv_s], ones16)
            iv_d = didx[pl.ds(j * 16, 16)]
            plsc.addupdate_scatter(hist_d, [iv_d], ones16)
            return carry

        lax.fori_loop(0, ept // 16, hbody, 0)
        pltpu.sync_copy(hist_s, sh_s.at[s])
        pltpu.sync_copy(hist_d, sh_d.at[s])
        plsc.subcore_barrier()
        r0 = s * rt

        def merge(table, out_ref):
            pltpu.sync_copy(table.at[0, pl.ds(r0, rt)], racc)

            def mbody(t, carry):
                pltpu.sync_copy(table.at[t, pl.ds(r0, rt)], tmp)

                def abody(j, c2):
                    sl = pl.ds(j * 16, 16)
                    racc[sl] = racc[sl] + tmp[sl]
                    return c2

                lax.fori_loop(0, rt // 16, abody, 0)
                return carry

            lax.fori_loop(1, NS, mbody, 0)
            pltpu.sync_copy(racc, out_ref)

        merge(sh_s, out_s.at[pl.ds(c * n + r0, rt)])
        merge(sh_d, out_d.at[pl.ds(c * n + r0, rt)])

    return deg


def _agg_kernel(n, e, h):
    ept = e // (NC * NS)
    ch = ept // K
    rt = n // NS

    @functools.partial(
        pl.kernel,
        out_type=jax.ShapeDtypeStruct((NC, n, h), jnp.float32),
        mesh=_mesh,
        scratch_types=[
            pltpu.VMEM((K,), jnp.int32),
            pltpu.VMEM((K,), jnp.int32),
            pltpu.VMEM((K,), jnp.int32),
            pltpu.VMEM((K,), jnp.int32),
            pltpu.VMEM((K,), jnp.int32),
            pltpu.VMEM((K,), jnp.int32),
            pltpu.VMEM((K, h), jnp.float32),
            pltpu.VMEM((K, h), jnp.float32),
            pltpu.SemaphoreType.DMA,
            pltpu.SemaphoreType.DMA,
            pltpu.SemaphoreType.DMA,
            pltpu.SemaphoreType.DMA,
            pltpu.SemaphoreType.DMA,
            pltpu.SemaphoreType.DMA,
            pltpu.VMEM_SHARED((n, h), jnp.float32),
        ],
    )
    def agg(h_hbm, src_hbm, dst_hbm, out,
            sidx0, sidx1, didx0, didx1, scp0, scp1, rows0, rows1,
            isem0, isem1, gsem0, gsem1, ssem0, ssem1, acc):
        c = lax.axis_index("c")
        s = lax.axis_index("s")
        z16 = jnp.zeros((16,), jnp.float32)
        for r in range(K):
            for j in range(h // 16):
                rows0[r, pl.ds(j * 16, 16)] = z16
        r0 = s * rt
        for j in range(rt // K):
            pltpu.sync_copy(rows0, acc.at[pl.ds(r0 + j * K, K)])
        plsc.subcore_barrier()
        ebase = (c * NS + s) * ept

        sidx = (sidx0, sidx1)
        didx = (didx0, didx1)
        scp = (scp0, scp1)
        rows = (rows0, rows1)
        isem = (isem0, isem1)
        gsem = (gsem0, gsem1)
        ssem = (ssem0, ssem1)

        pltpu.sync_copy(src_hbm.at[pl.ds(ebase, K)], sidx0)
        pltpu.sync_copy(dst_hbm.at[pl.ds(ebase, K)], didx0)
        pltpu.async_copy(h_hbm.at[sidx0], rows0, gsem0)
        pltpu.async_copy(src_hbm.at[pl.ds(ebase + K, K)], sidx1, isem1)
        pltpu.async_copy(dst_hbm.at[pl.ds(ebase + K, K)], didx1, isem1)

        def step(i, carry):
            def body(p, q):
                @pl.when(i + 1 < ch)
                def _():
                    b1 = ebase + (i + 1) * K
                    pltpu.make_async_copy(
                        src_hbm.at[pl.ds(b1, K)], sidx[q], isem[q]).wait()
                    pltpu.make_async_copy(
                        dst_hbm.at[pl.ds(b1, K)], didx[q], isem[q]).wait()

                    @pl.when(i >= 1)
                    def _():
                        pltpu.make_async_copy(
                            rows[q], acc.at[scp[q]], ssem[q]).wait()

                    pltpu.async_copy(h_hbm.at[sidx[q]], rows[q], gsem[q])

                pltpu.make_async_copy(h_hbm.at[sidx[p]], rows[p],
                                      gsem[p]).wait()
                for j in range(K // 16):
                    sl = pl.ds(j * 16, 16)
                    scp[p][sl] = didx[p][sl]
                pltpu.async_copy(rows[p], acc.at[scp[p]], ssem[p], add=True)

                @pl.when(i + 2 < ch)
                def _():
                    b2 = ebase + (i + 2) * K
                    pltpu.async_copy(src_hbm.at[pl.ds(b2, K)], sidx[p],
                                     isem[p])
                    pltpu.async_copy(dst_hbm.at[pl.ds(b2, K)], didx[p],
                                     isem[p])

            even = lax.rem(i, 2) == 0

            @pl.when(even)
            def _():
                body(0, 1)

            @pl.when(jnp.logical_not(even))
            def _():
                body(1, 0)

            return carry

        lax.fori_loop(0, ch, step, 0)
        lastp = (ch - 1) % 2
        pltpu.make_async_copy(rows[1 - lastp], acc.at[scp[1 - lastp]],
                              ssem[1 - lastp]).wait()
        pltpu.make_async_copy(rows[lastp], acc.at[scp[lastp]],
                              ssem[lastp]).wait()
        plsc.subcore_barrier()
        pltpu.sync_copy(acc.at[pl.ds(r0, rt)], out.at[c, pl.ds(r0, rt)])

    return agg


def _mm_body(x_ref, w_ref, d0_ref, d1_ref, h_ref):
    deg = d0_ref[...] + d1_ref[...]
    nsrc = lax.rsqrt(jnp.maximum(deg, 1.0))
    xs = x_ref[...] * nsrc[:, None]
    h_ref[...] = jnp.dot(xs, w_ref[...], preferred_element_type=jnp.float32)


def _fin_body(p_ref, d0_ref, d1_ref, b_ref, w1_ref, b1_ref, w2_ref, b2_ref,
              o_ref):
    n = d0_ref.shape[0]
    agg = p_ref[0, 0:n, :] + p_ref[1, 0:n, :]
    deg = d0_ref[...] + d1_ref[...]
    ndst = lax.rsqrt(jnp.maximum(deg, 1.0))
    hrelu = jnp.maximum(agg * ndst[:, None] + b_ref[...], 0.0)
    hg = jnp.sum(hrelu, axis=0, keepdims=True) * (1.0 / n)
    o1 = jnp.dot(hg, w1_ref[...], preferred_element_type=jnp.float32)
    o1 = o1 + b1_ref[...]
    o2 = jnp.dot(o1, w2_ref[...], preferred_element_type=jnp.float32)
    o_ref[...] = o2 + b2_ref[...]


def kernel(inputs, edge_index, W, b, W1, b1, W2, b2):
    n, d = inputs.shape
    h = W.shape[1]
    e = edge_index.shape[1]
    npad = -(-n // (NS * 16)) * (NS * 16)
    src = edge_index[0]
    dst = edge_index[1]

    deg_s, deg_d = _deg_kernel(npad, e)(src, dst)

    hmat = pl.pallas_call(
        _mm_body,
        out_shape=jax.ShapeDtypeStruct((n, h), jnp.float32),
    )(inputs, W, deg_s[:n], deg_s[npad:npad + n])

    grain = NC * NS * K
    ep = -(-e // grain) * grain
    if ep > e:
        pad = ep - e
        pad_i = jnp.arange(pad, dtype=jnp.int32)
        src_a = jnp.concatenate([src, pad_i % 8])
        dst_a = jnp.concatenate([dst, n + (pad_i % 16)])
    else:
        src_a, dst_a = src, dst

    parts = _agg_kernel(npad, ep, h)(hmat, src_a, dst_a)

    out = pl.pallas_call(
        _fin_body,
        out_shape=jax.ShapeDtypeStruct((1, W2.shape[1]), jnp.float32),
    )(parts, deg_d[:n], deg_d[npad:npad + n], b.reshape(1, h),
      W1, b1.reshape(1, -1), W2, b2.reshape(1, -1))
    return out


# speedup vs baseline: 10.8271x; 1.0418x over previous
import functools

import jax
import jax.numpy as jnp
from jax import lax
from jax.experimental import pallas as pl
from jax.experimental.pallas import tpu as pltpu
from jax.experimental.pallas import tpu_sc as plsc

NC = 2
NS = 16
NW = NC * NS
K = 128

_mesh = plsc.VectorSubcoreMesh(core_axis_name="c", subcore_axis_name="s")


def _zero_fill(ref, words):
    def body(j, carry):
        ref[pl.ds(j * 16, 16)] = jnp.zeros((16,), jnp.float32)
        return carry

    lax.fori_loop(0, words // 16, body, 0)


def _chunk_copy(ei_hbm, pad_hbm, g, creal, ebuf, sem):
    @pl.when(g < creal)
    def _():
        pltpu.async_copy(
            ei_hbm.at[pl.ds(0, 2), pl.ds(g * K, K)], ebuf, sem)

    @pl.when(g >= creal)
    def _():
        pltpu.async_copy(
            pad_hbm.at[pl.ds(0, 2), pl.ds((g - creal) * K, K)], ebuf, sem)


def _chunk_wait(ei_hbm, ebuf, sem):
    pltpu.make_async_copy(ei_hbm.at[pl.ds(0, 2), pl.ds(0, K)], ebuf,
                          sem).wait()


def _deg_kernel(n, e, ep):
    creal = e // K
    ch = ep // (NW * K)
    rt = n // NS

    @functools.partial(
        pl.kernel,
        out_type=(
            jax.ShapeDtypeStruct((NC * n,), jnp.float32),
            jax.ShapeDtypeStruct((NC * n,), jnp.float32),
        ),
        mesh=_mesh,
        compiler_params=pltpu.CompilerParams(needs_layout_passes=False),
        scratch_types=[
            pltpu.VMEM((2, K), jnp.int32),
            pltpu.VMEM((2, K), jnp.int32),
            pltpu.VMEM((n,), jnp.float32),
            pltpu.VMEM((n,), jnp.float32),
            pltpu.VMEM((rt,), jnp.float32),
            pltpu.VMEM((rt,), jnp.float32),
            pltpu.SemaphoreType.DMA,
            pltpu.SemaphoreType.DMA,
            pltpu.VMEM_SHARED((NS, n), jnp.float32),
            pltpu.VMEM_SHARED((NS, n), jnp.float32),
        ],
    )
    def deg(ei_hbm, pad_hbm, out_s, out_d,
            ebuf0, ebuf1, hist_s, hist_d, racc, tmp, isem0, isem1,
            sh_s, sh_d):
        c = lax.axis_index("c")
        s = lax.axis_index("s")
        wid = c * NS + s
        ebuf = (ebuf0, ebuf1)
        isem = (isem0, isem1)
        _chunk_copy(ei_hbm, pad_hbm, wid, creal, ebuf0, isem0)
        _chunk_copy(ei_hbm, pad_hbm, wid + NW, creal, ebuf1, isem1)
        _zero_fill(hist_s, n)
        _zero_fill(hist_d, n)
        ones16 = jnp.ones((16,), jnp.float32)

        def step(j, carry):
            def body(p):
                _chunk_wait(ei_hbm, ebuf[p], isem[p])
                for jj in range(K // 16):
                    sl = pl.ds(jj * 16, 16)
                    plsc.addupdate_scatter(hist_s, [ebuf[p][0, sl]], ones16)
                    plsc.addupdate_scatter(hist_d, [ebuf[p][1, sl]], ones16)

                @pl.when(j + 2 < ch)
                def _():
                    _chunk_copy(ei_hbm, pad_hbm, wid + (j + 2) * NW, creal,
                                ebuf[p], isem[p])

            even = lax.rem(j, 2) == 0

            @pl.when(even)
            def _():
                body(0)

            @pl.when(jnp.logical_not(even))
            def _():
                body(1)

            return carry

        lax.fori_loop(0, ch, step, 0)
        pltpu.sync_copy(hist_s, sh_s.at[s])
        pltpu.sync_copy(hist_d, sh_d.at[s])
        plsc.subcore_barrier()
        r0 = s * rt

        def merge(table, out_ref):
            pltpu.sync_copy(table.at[0, pl.ds(r0, rt)], racc)

            def mbody(t, carry):
                pltpu.sync_copy(table.at[t, pl.ds(r0, rt)], tmp)

                def abody(jj, c2):
                    sl = pl.ds(jj * 16, 16)
                    racc[sl] = racc[sl] + tmp[sl]
                    return c2

                lax.fori_loop(0, rt // 16, abody, 0)
                return carry

            lax.fori_loop(1, NS, mbody, 0)
            pltpu.sync_copy(racc, out_ref)

        merge(sh_s, out_s.at[pl.ds(c * n + r0, rt)])
        merge(sh_d, out_d.at[pl.ds(c * n + r0, rt)])

    return deg


def _agg_kernel(n, e, ep, h):
    creal = e // K
    ch = ep // (NW * K)
    rt = n // NS

    @functools.partial(
        pl.kernel,
        out_type=jax.ShapeDtypeStruct((NC, n, h), jnp.float32),
        mesh=_mesh,
        compiler_params=pltpu.CompilerParams(needs_layout_passes=False),
        scratch_types=[
            pltpu.VMEM((2, K), jnp.int32),
            pltpu.VMEM((2, K), jnp.int32),
            pltpu.VMEM((K,), jnp.int32),
            pltpu.VMEM((K,), jnp.int32),
            pltpu.VMEM((K, h), jnp.float32),
            pltpu.VMEM((K, h), jnp.float32),
            pltpu.SemaphoreType.DMA,
            pltpu.SemaphoreType.DMA,
            pltpu.SemaphoreType.DMA,
            pltpu.SemaphoreType.DMA,
            pltpu.SemaphoreType.DMA,
            pltpu.SemaphoreType.DMA,
            pltpu.VMEM_SHARED((n, h), jnp.float32),
        ],
    )
    def agg(h_hbm, ei_hbm, pad_hbm, out,
            ebuf0, ebuf1, scp0, scp1, rows0, rows1,
            isem0, isem1, gsem0, gsem1, ssem0, ssem1, acc):
        c = lax.axis_index("c")
        s = lax.axis_index("s")
        wid = c * NS + s
        z16 = jnp.zeros((16,), jnp.float32)
        for r in range(K):
            for j in range(h // 16):
                rows0[r, pl.ds(j * 16, 16)] = z16
        r0 = s * rt
        for j in range(rt // K):
            pltpu.sync_copy(rows0, acc.at[pl.ds(r0 + j * K, K)])
        plsc.subcore_barrier()

        ebuf = (ebuf0, ebuf1)
        scp = (scp0, scp1)
        rows = (rows0, rows1)
        isem = (isem0, isem1)
        gsem = (gsem0, gsem1)
        ssem = (ssem0, ssem1)

        _chunk_copy(ei_hbm, pad_hbm, wid, creal, ebuf0, isem0)
        _chunk_copy(ei_hbm, pad_hbm, wid + NW, creal, ebuf1, isem1)
        _chunk_wait(ei_hbm, ebuf0, isem0)
        pltpu.async_copy(h_hbm.at[ebuf0.at[0]], rows0, gsem0)

        def step(j, carry):
            def body(p, q):
                @pl.when(j + 1 < ch)
                def _():
                    _chunk_wait(ei_hbm, ebuf[q], isem[q])

                    @pl.when(j >= 1)
                    def _():
                        pltpu.make_async_copy(
                            rows[q], acc.at[scp[q]], ssem[q]).wait()

                    pltpu.async_copy(h_hbm.at[ebuf[q].at[0]], rows[q],
                                     gsem[q])

                pltpu.make_async_copy(h_hbm.at[ebuf[p].at[0]], rows[p],
                                      gsem[p]).wait()
                for jj in range(K // 16):
                    sl = pl.ds(jj * 16, 16)
                    scp[p][sl] = ebuf[p][1, sl]
                pltpu.async_copy(rows[p], acc.at[scp[p]], ssem[p], add=True)

                @pl.when(j + 2 < ch)
                def _():
                    _chunk_copy(ei_hbm, pad_hbm, wid + (j + 2) * NW, creal,
                                ebuf[p], isem[p])

            even = lax.rem(j, 2) == 0

            @pl.when(even)
            def _():
                body(0, 1)

            @pl.when(jnp.logical_not(even))
            def _():
                body(1, 0)

            return carry

        lax.fori_loop(0, ch, step, 0)
        lastp = (ch - 1) % 2
        pltpu.make_async_copy(rows[1 - lastp], acc.at[scp[1 - lastp]],
                              ssem[1 - lastp]).wait()
        pltpu.make_async_copy(rows[lastp], acc.at[scp[lastp]],
                              ssem[lastp]).wait()
        plsc.subcore_barrier()
        pltpu.sync_copy(acc.at[pl.ds(r0, rt)], out.at[c, pl.ds(r0, rt)])

    return agg


def _mm_body(x_ref, w_ref, d0_ref, d1_ref, h_ref):
    n = x_ref.shape[0]
    npad = h_ref.shape[0]
    deg = d0_ref[...] + d1_ref[...]
    nsrc = lax.rsqrt(jnp.maximum(deg, 1.0))
    xs = x_ref[...] * nsrc[:, None]
    h_ref[0:n, :] = jnp.dot(xs, w_ref[...],
                            preferred_element_type=jnp.float32)
    h_ref[n:npad, :] = jnp.zeros((npad - n, h_ref.shape[1]), jnp.float32)


def _fin_body(p_ref, d0_ref, d1_ref, b_ref, w1_ref, b1_ref, w2_ref, b2_ref,
              o_ref):
    n = d0_ref.shape[0]
    agg = p_ref[0, 0:n, :] + p_ref[1, 0:n, :]
    deg = d0_ref[...] + d1_ref[...]
    ndst = lax.rsqrt(jnp.maximum(deg, 1.0))
    hrelu = jnp.maximum(agg * ndst[:, None] + b_ref[...], 0.0)
    hg = jnp.sum(hrelu, axis=0, keepdims=True) * (1.0 / n)
    o1 = jnp.dot(hg, w1_ref[...], preferred_element_type=jnp.float32)
    o1 = o1 + b1_ref[...]
    o2 = jnp.dot(o1, w2_ref[...], preferred_element_type=jnp.float32)
    o_ref[...] = o2 + b2_ref[...]


def kernel(inputs, edge_index, W, b, W1, b1, W2, b2):
    n, d = inputs.shape
    h = W.shape[1]
    e = edge_index.shape[1]
    npad = -(-n // (NS * 16)) * (NS * 16)
    grain = NW * K
    ep = -(-e // grain) * grain
    pade = ep - e
    pad_i = jnp.arange(max(pade, grain), dtype=jnp.int32)[:pade]
    pad_ei = jnp.stack([n + (pad_i * 7) % (npad - n),
                        n + pad_i % (npad - n)])

    deg_s, deg_d = _deg_kernel(npad, e, ep)(edge_index, pad_ei)

    hmat = pl.pallas_call(
        _mm_body,
        out_shape=jax.ShapeDtypeStruct((npad, h), jnp.float32),
    )(inputs, W, deg_s[:n], deg_s[npad:npad + n])

    parts = _agg_kernel(npad, e, ep, h)(hmat, edge_index, pad_ei)

    out = pl.pallas_call(
        _fin_body,
        out_shape=jax.ShapeDtypeStruct((1, W2.shape[1]), jnp.float32),
    )(parts, deg_d[:n], deg_d[npad:npad + n], b.reshape(1, h),
      W1, b1.reshape(1, -1), W2, b2.reshape(1, -1))
    return out


# speedup vs baseline: 11.9281x; 1.1017x over previous
import functools

import jax
import jax.numpy as jnp
from jax import lax
from jax.experimental import pallas as pl
from jax.experimental.pallas import tpu as pltpu
from jax.experimental.pallas import tpu_sc as plsc

NC = 2
NS = 16
NW = NC * NS
K = 128

_mesh = plsc.VectorSubcoreMesh(core_axis_name="c", subcore_axis_name="s")


def _zero_fill(ref, words):
    def body(j, carry):
        ref[pl.ds(j * 16, 16)] = jnp.zeros((16,), jnp.float32)
        return carry

    lax.fori_loop(0, words // 16, body, 0)


def _chunk_copy(ei_hbm, pad_hbm, g, creal, ebuf, sem):
    @pl.when(g < creal)
    def _():
        pltpu.async_copy(
            ei_hbm.at[pl.ds(0, 2), pl.ds(g * K, K)], ebuf, sem)

    @pl.when(g >= creal)
    def _():
        pltpu.async_copy(
            pad_hbm.at[pl.ds(0, 2), pl.ds((g - creal) * K, K)], ebuf, sem)


def _chunk_wait(ei_hbm, ebuf, sem):
    pltpu.make_async_copy(ei_hbm.at[pl.ds(0, 2), pl.ds(0, K)], ebuf,
                          sem).wait()


def _deg_kernel(n, e, ep):
    creal = e // K
    ch = ep // (NW * K)
    rt = n // NS

    ept = ch * K
    bt = e // ept
    rem = e - bt * ept

    @functools.partial(
        pl.kernel,
        out_type=(
            jax.ShapeDtypeStruct((NC * n,), jnp.float32),
            jax.ShapeDtypeStruct((NC * n,), jnp.float32),
        ),
        mesh=_mesh,
        compiler_params=pltpu.CompilerParams(needs_layout_passes=False),
        scratch_types=[
            pltpu.VMEM((2, ch * K), jnp.int32),
            pltpu.VMEM((n,), jnp.float32),
            pltpu.VMEM((n,), jnp.float32),
            pltpu.VMEM((rt,), jnp.float32),
            pltpu.VMEM((rt,), jnp.float32),
            pltpu.SemaphoreType.DMA,
            pltpu.VMEM_SHARED((NS, n), jnp.float32),
            pltpu.VMEM_SHARED((NS, n), jnp.float32),
        ],
    )
    def deg(ei_hbm, pad_hbm, out_s, out_d,
            ebuf, hist_s, hist_d, racc, tmp, isem, sh_s, sh_d):
        c = lax.axis_index("c")
        s = lax.axis_index("s")
        wid = c * NS + s
        if bt > 0:
            @pl.when(wid < bt)
            def _():
                pltpu.async_copy(
                    ei_hbm.at[pl.ds(0, 2), pl.ds(wid * ept, ept)], ebuf,
                    isem)

        if rem > 0:
            @pl.when(wid == bt)
            def _():
                pltpu.async_copy(
                    ei_hbm.at[pl.ds(0, 2), pl.ds(bt * ept, rem)],
                    ebuf.at[pl.ds(0, 2), pl.ds(0, rem)], isem)
                pltpu.async_copy(
                    pad_hbm.at[pl.ds(0, 2), pl.ds(0, ept - rem)],
                    ebuf.at[pl.ds(0, 2), pl.ds(rem, ept - rem)], isem)

        if bt + 1 < NW:
            @pl.when(wid > bt)
            def _():
                pltpu.async_copy(
                    pad_hbm.at[pl.ds(0, 2),
                               pl.ds(wid * ept - e, ept)], ebuf, isem)

        _zero_fill(hist_s, n)
        _zero_fill(hist_d, n)
        pltpu.make_async_copy(
            ei_hbm.at[pl.ds(0, 2), pl.ds(0, ept)], ebuf, isem).wait()
        ones16 = jnp.ones((16,), jnp.float32)

        def step(j, carry):
            sl = pl.ds(j * 16, 16)
            plsc.addupdate_scatter(hist_s, [ebuf[0, sl]], ones16)
            plsc.addupdate_scatter(hist_d, [ebuf[1, sl]], ones16)
            return carry

        lax.fori_loop(0, ept // 16, step, 0)
        pltpu.sync_copy(hist_s, sh_s.at[s])
        pltpu.sync_copy(hist_d, sh_d.at[s])
        plsc.subcore_barrier()
        r0 = s * rt

        def merge(table, out_ref):
            pltpu.sync_copy(table.at[0, pl.ds(r0, rt)], racc)

            def mbody(t, carry):
                pltpu.sync_copy(table.at[t, pl.ds(r0, rt)], tmp)

                def abody(jj, c2):
                    sl = pl.ds(jj * 16, 16)
                    racc[sl] = racc[sl] + tmp[sl]
                    return c2

                lax.fori_loop(0, rt // 16, abody, 0)
                return carry

            lax.fori_loop(1, NS, mbody, 0)
            pltpu.sync_copy(racc, out_ref)

        merge(sh_s, out_s.at[pl.ds(c * n + r0, rt)])
        merge(sh_d, out_d.at[pl.ds(c * n + r0, rt)])

    return deg


def _agg_kernel(n, e, ep, h):
    creal = e // K
    ch = ep // (NW * K)
    rt = n // NS

    @functools.partial(
        pl.kernel,
        out_type=jax.ShapeDtypeStruct((NC, n, h), jnp.float32),
        mesh=_mesh,
        compiler_params=pltpu.CompilerParams(needs_layout_passes=False),
        scratch_types=[
            pltpu.VMEM((2, K), jnp.int32),
            pltpu.VMEM((2, K), jnp.int32),
            pltpu.VMEM((K,), jnp.int32),
            pltpu.VMEM((K,), jnp.int32),
            pltpu.VMEM((K, h), jnp.float32),
            pltpu.VMEM((K, h), jnp.float32),
            pltpu.SemaphoreType.DMA,
            pltpu.SemaphoreType.DMA,
            pltpu.SemaphoreType.DMA,
            pltpu.SemaphoreType.DMA,
            pltpu.SemaphoreType.DMA,
            pltpu.SemaphoreType.DMA,
            pltpu.VMEM_SHARED((n, h), jnp.float32),
        ],
    )
    def agg(h_hbm, ei_hbm, pad_hbm, out,
            ebuf0, ebuf1, scp0, scp1, rows0, rows1,
            isem0, isem1, gsem0, gsem1, ssem0, ssem1, acc):
        c = lax.axis_index("c")
        s = lax.axis_index("s")
        wid = c * NS + s
        z16 = jnp.zeros((16,), jnp.float32)
        for r in range(K):
            for j in range(h // 16):
                rows0[r, pl.ds(j * 16, 16)] = z16
        r0 = s * rt
        for j in range(rt // K):
            pltpu.sync_copy(rows0, acc.at[pl.ds(r0 + j * K, K)])
        plsc.subcore_barrier()

        ebuf = (ebuf0, ebuf1)
        scp = (scp0, scp1)
        rows = (rows0, rows1)
        isem = (isem0, isem1)
        gsem = (gsem0, gsem1)
        ssem = (ssem0, ssem1)

        _chunk_copy(ei_hbm, pad_hbm, wid, creal, ebuf0, isem0)
        _chunk_copy(ei_hbm, pad_hbm, wid + NW, creal, ebuf1, isem1)
        _chunk_wait(ei_hbm, ebuf0, isem0)
        pltpu.async_copy(h_hbm.at[ebuf0.at[0]], rows0, gsem0)

        def step(j, carry):
            def body(p, q):
                @pl.when(j + 1 < ch)
                def _():
                    _chunk_wait(ei_hbm, ebuf[q], isem[q])

                    @pl.when(j >= 1)
                    def _():
                        pltpu.make_async_copy(
                            rows[q], acc.at[scp[q]], ssem[q]).wait()

                    pltpu.async_copy(h_hbm.at[ebuf[q].at[0]], rows[q],
                                     gsem[q])

                pltpu.make_async_copy(h_hbm.at[ebuf[p].at[0]], rows[p],
                                      gsem[p]).wait()
                for jj in range(K // 16):
                    sl = pl.ds(jj * 16, 16)
                    scp[p][sl] = ebuf[p][1, sl]
                pltpu.async_copy(rows[p], acc.at[scp[p]], ssem[p], add=True)

                @pl.when(j + 2 < ch)
                def _():
                    _chunk_copy(ei_hbm, pad_hbm, wid + (j + 2) * NW, creal,
                                ebuf[p], isem[p])

            even = lax.rem(j, 2) == 0

            @pl.when(even)
            def _():
                body(0, 1)

            @pl.when(jnp.logical_not(even))
            def _():
                body(1, 0)

            return carry

        lax.fori_loop(0, ch, step, 0)
        lastp = (ch - 1) % 2
        pltpu.make_async_copy(rows[1 - lastp], acc.at[scp[1 - lastp]],
                              ssem[1 - lastp]).wait()
        pltpu.make_async_copy(rows[lastp], acc.at[scp[lastp]],
                              ssem[lastp]).wait()
        plsc.subcore_barrier()
        pltpu.sync_copy(acc.at[pl.ds(r0, rt)], out.at[c, pl.ds(r0, rt)])

    return agg


def _mm_body(x_ref, w_ref, d0_ref, d1_ref, h_ref):
    n = x_ref.shape[0]
    npad = h_ref.shape[0]
    deg = d0_ref[...] + d1_ref[...]
    nsrc = lax.rsqrt(jnp.maximum(deg, 1.0))
    xs = x_ref[...] * nsrc[:, None]
    h_ref[0:n, :] = jnp.dot(xs, w_ref[...],
                            preferred_element_type=jnp.float32)
    h_ref[n:npad, :] = jnp.zeros((npad - n, h_ref.shape[1]), jnp.float32)


def _fin_body(p_ref, d0_ref, d1_ref, b_ref, w1_ref, b1_ref, w2_ref, b2_ref,
              o_ref):
    n = d0_ref.shape[0]
    agg = p_ref[0, 0:n, :] + p_ref[1, 0:n, :]
    deg = d0_ref[...] + d1_ref[...]
    ndst = lax.rsqrt(jnp.maximum(deg, 1.0))
    hrelu = jnp.maximum(agg * ndst[:, None] + b_ref[...], 0.0)
    hg = jnp.sum(hrelu, axis=0, keepdims=True) * (1.0 / n)
    o1 = jnp.dot(hg, w1_ref[...], preferred_element_type=jnp.float32)
    o1 = o1 + b1_ref[...]
    o2 = jnp.dot(o1, w2_ref[...], preferred_element_type=jnp.float32)
    o_ref[...] = o2 + b2_ref[...]


def kernel(inputs, edge_index, W, b, W1, b1, W2, b2):
    n, d = inputs.shape
    h = W.shape[1]
    e = edge_index.shape[1]
    npad = -(-n // (NS * 16)) * (NS * 16)
    grain = NW * K
    ep = -(-e // grain) * grain
    pade = ep - e
    pad_i = jnp.arange(max(pade, grain), dtype=jnp.int32)[:pade]
    pad_ei = jnp.stack([n + (pad_i * 7) % (npad - n),
                        n + pad_i % (npad - n)])

    deg_s, deg_d = _deg_kernel(npad, e, ep)(edge_index, pad_ei)

    hmat = pl.pallas_call(
        _mm_body,
        out_shape=jax.ShapeDtypeStruct((npad, h), jnp.float32),
    )(inputs, W, deg_s[:n], deg_s[npad:npad + n])

    parts = _agg_kernel(npad, e, ep, h)(hmat, edge_index, pad_ei)

    out = pl.pallas_call(
        _fin_body,
        out_shape=jax.ShapeDtypeStruct((1, W2.shape[1]), jnp.float32),
    )(parts, deg_d[:n], deg_d[npad:npad + n], b.reshape(1, h),
      W1, b1.reshape(1, -1), W2, b2.reshape(1, -1))
    return out
